# baseline (speedup 1.0000x reference)
import functools

import jax
import jax.numpy as jnp
from jax import lax
from jax.experimental import pallas as pl
from jax.experimental.pallas import tpu as pltpu
from jax.experimental.pallas import tpu_sc as plsc

N = 10000
D = 128
DH = 64
NG = 64
NCLS = 10
NP = 10112
NB = 79
E = 320000
CH = 128
NSC = 2
NTS = 16
NTW = NSC * NTS
CPT = 80
PT = CPT // 2
EPT = CPT * CH
EPAD = NTW * EPT
RPT = NP // NTS

_mesh = plsc.VectorSubcoreMesh(
    core_axis_name="c", subcore_axis_name="s", num_cores=NSC,
    num_subcores=NTS)


def _hist_body(dstp_hbm, hist_hbm, idx_v, ones_b, zbuf, shared_hist):
    c = lax.axis_index("c")
    s = lax.axis_index("s")
    w = c * NTS + s
    pltpu.sync_copy(dstp_hbm.at[w], idx_v)

    zero16 = jnp.zeros((16,), jnp.float32)
    one16 = jnp.ones((16,), jnp.float32)

    def _z(i, _):
        zbuf[i, :] = zero16
        return 0

    lax.fori_loop(0, 158, _z, 0)

    def _o(i, _):
        ones_b[i, :] = one16
        return 0

    lax.fori_loop(0, CH, _o, 0)

    for k in range(4):
        pltpu.sync_copy(zbuf, shared_hist.at[pl.ds(s * RPT + k * 158, 158)])
    plsc.subcore_barrier()

    def _chunk(j, _):
        pltpu.sync_copy(ones_b, shared_hist.at[idx_v.at[j]], add=True)
        return 0

    lax.fori_loop(0, CPT, _chunk, 0)
    plsc.subcore_barrier()

    pltpu.sync_copy(shared_hist.at[pl.ds(s * RPT, RPT)],
                    hist_hbm.at[c].at[pl.ds(s * RPT, RPT)])


_hist_call = functools.partial(
    pl.kernel,
    out_type=jax.ShapeDtypeStruct((NSC, NP, 16), jnp.float32),
    mesh=_mesh,
    scratch_types=[
        pltpu.VMEM((CPT, CH), jnp.int32),
        pltpu.VMEM((CH, 16), jnp.float32),
        pltpu.VMEM((158, 16), jnp.float32),
        pltpu.VMEM_SHARED((NP, 16), jnp.float32),
    ],
)(_hist_body)


def _scat_body(u_hbm, z_hbm, srcp_hbm, dstp_hbm, s_hbm, dst_v, ring, gbuf0,
               gbuf1, shared_out, semg, sema, semb):
    c = lax.axis_index("c")
    s = lax.axis_index("s")
    w = c * NTS + s
    pltpu.sync_copy(dstp_hbm.at[w], dst_v)
    pltpu.sync_copy(z_hbm.at[pl.ds(s * RPT, RPT)],
                    shared_out.at[pl.ds(s * RPT, RPT)])
    sp = srcp_hbm.at[w]
    pltpu.sync_copy(sp.at[pl.ds(0, 2)], ring.at[pl.ds(0, 2)])
    pltpu.async_copy(sp.at[pl.ds(2, 2)], ring.at[pl.ds(2, 2)], semb)
    pltpu.async_copy(sp.at[pl.ds(4, 2)], ring.at[pl.ds(4, 2)], sema)
    plsc.subcore_barrier()

    pltpu.async_copy(u_hbm.at[ring.at[0]], gbuf0, semg).wait()

    def _half(p, wsem):
        j0 = 2 * p
        j1 = j0 + 1
        j2 = jnp.where(j0 + 2 < CPT, j0 + 2, 0)
        pltpu.make_async_copy(
            sp.at[pl.ds(0, 2)], ring.at[pl.ds(0, 2)], wsem).wait()
        jn = jnp.where(j0 + 6 < CPT, j0 + 6, 0)
        pltpu.async_copy(sp.at[pl.ds(jn, 2)], ring.at[pl.ds(jn & 7, 2)], wsem)
        cp1 = pltpu.async_copy(u_hbm.at[ring.at[j1 & 7]], gbuf1, semg)
        pltpu.sync_copy(gbuf0, shared_out.at[dst_v.at[j0]], add=True)
        cp1.wait()
        cp2 = pltpu.async_copy(u_hbm.at[ring.at[j2 & 7]], gbuf0, semg)
        pltpu.sync_copy(gbuf1, shared_out.at[dst_v.at[j1]], add=True)
        cp2.wait()

    def _pairs(t, _):
        _half(2 * t, semb)
        _half(2 * t + 1, sema)
        return 0

    lax.fori_loop(0, PT // 2, _pairs, 0)
    plsc.subcore_barrier()

    pltpu.sync_copy(shared_out.at[pl.ds(s * RPT, RPT)],
                    s_hbm.at[c].at[pl.ds(s * RPT, RPT)])


_scat_call = functools.partial(
    pl.kernel,
    out_type=jax.ShapeDtypeStruct((NSC, NP, D), jnp.float32),
    mesh=_mesh,
    scratch_types=[
        pltpu.VMEM((CPT, CH), jnp.int32),
        pltpu.VMEM((8, CH), jnp.int32),
        pltpu.VMEM((CH, D), jnp.float32),
        pltpu.VMEM((CH, D), jnp.float32),
        pltpu.VMEM_SHARED((NP, D), jnp.float32),
        pltpu.SemaphoreType.DMA,
        pltpu.SemaphoreType.DMA,
        pltpu.SemaphoreType.DMA,
    ],
)(_scat_body)


def _tc1_body(hist_ref, x_ref, dinv_ref, u_ref):
    deg = hist_ref[0, :, 0:1] + hist_ref[1, :, 0:1] + 1.0
    dinv = lax.rsqrt(deg)
    dinv_ref[...] = dinv
    u_ref[...] = x_ref[...] * dinv


def _tc1(hist, xpad):
    return pl.pallas_call(
        _tc1_body,
        grid=(NB,),
        in_specs=[
            pl.BlockSpec((NSC, 128, 16), lambda r: (0, r, 0)),
            pl.BlockSpec((128, D), lambda r: (r, 0)),
        ],
        out_specs=[
            pl.BlockSpec((128, 1), lambda r: (r, 0)),
            pl.BlockSpec((128, D), lambda r: (r, 0)),
        ],
        out_shape=[
            jax.ShapeDtypeStruct((NP, 1), jnp.float32),
            jax.ShapeDtypeStruct((NP, D), jnp.float32),
        ],
    )(hist, xpad)


_SP_SPEC = pl.BlockSpec((NSC, 128, D), lambda r: (0, r, 0))
_ROW_SPEC = pl.BlockSpec((128, D), lambda r: (r, 0))
_DINV_SPEC = pl.BlockSpec((128, 1), lambda r: (r, 0))
_W_SPEC = pl.BlockSpec((D, D), lambda r: (0, 0))
_B_SPEC = pl.BlockSpec((1, D), lambda r: (0, 0))


def _tc2_body(sp_ref, u_ref, dinv_ref, o_ref):
    dinv = dinv_ref[...]
    d2 = dinv * dinv
    o_ref[...] = (sp_ref[0] + sp_ref[1] + u_ref[...]) * d2


def _tc2(sp, u, dinv):
    return pl.pallas_call(
        _tc2_body,
        grid=(NB,),
        in_specs=[_SP_SPEC, _ROW_SPEC, _DINV_SPEC],
        out_specs=_ROW_SPEC,
        out_shape=jax.ShapeDtypeStruct((NP, D), jnp.float32),
    )(sp, u, dinv)


def _tc3_body(sp_ref, u_ref, dinv_ref, Wsg_ref, bsg_ref, Wg1_ref, v_ref):
    dinv = dinv_ref[...]
    t = (sp_ref[0] + sp_ref[1] + u_ref[...]) * dinv
    h = jnp.maximum(
        jnp.dot(t, Wsg_ref[...], preferred_element_type=jnp.float32)
        + bsg_ref[...], 0.0)
    v_ref[...] = jnp.dot(
        h, Wg1_ref[...], preferred_element_type=jnp.float32) * dinv


def _tc3(sp, u2, dinv, W_sg, b_sg, W_g1):
    return pl.pallas_call(
        _tc3_body,
        grid=(NB,),
        in_specs=[_SP_SPEC, _ROW_SPEC, _DINV_SPEC, _W_SPEC, _B_SPEC, _W_SPEC],
        out_specs=_ROW_SPEC,
        out_shape=jax.ShapeDtypeStruct((NP, D), jnp.float32),
    )(sp, u2, dinv, W_sg, b_sg, W_g1)


def _tc45_body(sp_ref, v_ref, dinv_ref, b_ref, W_ref, o_ref):
    dinv = dinv_ref[...]
    t = (sp_ref[0] + sp_ref[1] + v_ref[...]) * dinv
    h = jnp.maximum(t + b_ref[...], 0.0)
    o_ref[...] = jnp.dot(
        h, W_ref[...], preferred_element_type=jnp.float32) * dinv


def _tc45(sp, v, dinv, b_prev, W_next):
    return pl.pallas_call(
        _tc45_body,
        grid=(NB,),
        in_specs=[_SP_SPEC, _ROW_SPEC, _DINV_SPEC, _B_SPEC, _W_SPEC],
        out_specs=_ROW_SPEC,
        out_shape=jax.ShapeDtypeStruct((NP, D), jnp.float32),
    )(sp, v, dinv, b_prev, W_next)


def _tc6_body(sp_ref, v_ref, dinv_ref, bg3_ref, batch_ref, W1_ref, b1_ref,
              W2_ref, b2_ref, W3_ref, b3_ref, out_ref, g_acc):
    r = pl.program_id(0)

    @pl.when(r == 0)
    def _():
        g_acc[...] = jnp.zeros_like(g_acc)

    dinv = dinv_ref[...]
    t = (sp_ref[0] + sp_ref[1] + v_ref[...]) * dinv
    h = jnp.maximum(t + bg3_ref[...], 0.0)
    gi = lax.broadcasted_iota(jnp.int32, (NG, 1), 0)
    oh = (batch_ref[0] == gi).astype(jnp.float32)
    g_acc[...] += jnp.dot(oh, h, preferred_element_type=jnp.float32)

    @pl.when(r == NB - 1)
    def _():
        g = g_acc[...]
        g1 = jnp.maximum(
            jnp.dot(g, W1_ref[...], preferred_element_type=jnp.float32)
            + b1_ref[...], 0.0)
        g2 = jnp.maximum(
            jnp.dot(g1, W2_ref[...], preferred_element_type=jnp.float32)
            + b2_ref[...], 0.0)
        out_ref[...] = (
            jnp.dot(g2, W3_ref[...], preferred_element_type=jnp.float32)
            + b3_ref[...])


def _tc6(sp, v3, dinv, b_g3, batchp, W1, b1, W2, b2, W3, b3):
    return pl.pallas_call(
        _tc6_body,
        grid=(NB,),
        in_specs=[
            _SP_SPEC, _ROW_SPEC, _DINV_SPEC, _B_SPEC,
            pl.BlockSpec((1, 1, 128), lambda r: (r, 0, 0)),
            _W_SPEC, _B_SPEC,
            pl.BlockSpec((D, DH), lambda r: (0, 0)),
            pl.BlockSpec((1, DH), lambda r: (0, 0)),
            pl.BlockSpec((DH, NCLS), lambda r: (0, 0)),
            pl.BlockSpec((1, NCLS), lambda r: (0, 0)),
        ],
        out_specs=pl.BlockSpec((NG, NCLS), lambda r: (0, 0)),
        out_shape=jax.ShapeDtypeStruct((NG, NCLS), jnp.float32),
        scratch_shapes=[pltpu.VMEM((NG, D), jnp.float32)],
    )(sp, v3, dinv, b_g3, batchp, W1, b1, W2, b2, W3, b3)


def kernel(x, edge_index, batch, W_sg, b_sg, W_g1, b_g1, W_g2, b_g2,
           W_g3, b_g3, W_fc1, b_fc1, W_fc2, b_fc2, W_fc3, b_fc3):
    src = edge_index[0]
    dst = edge_index[1]
    pad = EPAD - E
    srcp = jnp.concatenate(
        [src, jnp.zeros((pad,), jnp.int32)]).reshape(NTW, CPT, CH)
    dstp = jnp.concatenate(
        [dst, jnp.full((pad,), N, jnp.int32)]).reshape(NTW, CPT, CH)
    xpad = jnp.pad(x, ((0, NP - N), (0, 0)))
    batchp = jnp.pad(batch, (0, NP - N),
                     constant_values=NG).reshape(NB, 1, 128)

    zeros = jnp.zeros((NP, D), jnp.float32)

    hist = _hist_call(dstp)
    dinv, u = _tc1(hist, xpad)
    sp1 = _scat_call(u, zeros, srcp, dstp)
    u2 = _tc2(sp1, u, dinv)
    sp2 = _scat_call(u2, zeros, srcp, dstp)
    v1 = _tc3(sp2, u2, dinv, W_sg, b_sg.reshape(1, D), W_g1)
    sp3 = _scat_call(v1, zeros, srcp, dstp)
    v2 = _tc45(sp3, v1, dinv, b_g1.reshape(1, D), W_g2)
    sp4 = _scat_call(v2, zeros, srcp, dstp)
    v3 = _tc45(sp4, v2, dinv, b_g2.reshape(1, D), W_g3)
    sp5 = _scat_call(v3, zeros, srcp, dstp)
    return _tc6(sp5, v3, dinv, b_g3.reshape(1, D), batchp,
                W_fc1, b_fc1.reshape(1, D), W_fc2, b_fc2.reshape(1, DH),
                W_fc3, b_fc3.reshape(1, NCLS))


# speedup vs baseline: 1.4930x; 1.4930x over previous
import functools

import jax
import jax.numpy as jnp
from jax import lax
from jax.experimental import pallas as pl
from jax.experimental.pallas import tpu as pltpu
from jax.experimental.pallas import tpu_sc as plsc

N = 10000
D = 128
DH = 64
NG = 64
NCLS = 10
NP = 10112
NB = 79
E = 320000
CH = 128
NSC = 2
NTS = 16
NTW = NSC * NTS
CPT = 79
EPT = CPT * CH
EPAD = NTW * EPT
RPT = NP // NTS

_mesh = plsc.VectorSubcoreMesh(
    core_axis_name="c", subcore_axis_name="s", num_cores=NSC,
    num_subcores=NTS)


def _hist_body(dstp_hbm, hist_hbm, idx_v, ones_b, zbuf, shared_hist):
    c = lax.axis_index("c")
    s = lax.axis_index("s")
    w = c * NTS + s
    pltpu.sync_copy(dstp_hbm.at[w], idx_v)

    zero16 = jnp.zeros((16,), jnp.float32)
    one16 = jnp.ones((16,), jnp.float32)

    def _z(i, _):
        zbuf[i, :] = zero16
        return 0

    lax.fori_loop(0, 158, _z, 0)

    def _o(i, _):
        ones_b[i, :] = one16
        return 0

    lax.fori_loop(0, CH, _o, 0)

    for k in range(4):
        pltpu.sync_copy(zbuf, shared_hist.at[pl.ds(s * RPT + k * 158, 158)])
    plsc.subcore_barrier()

    def _chunk(j, _):
        pltpu.sync_copy(ones_b, shared_hist.at[idx_v.at[j]], add=True)
        return 0

    lax.fori_loop(0, CPT, _chunk, 0)
    plsc.subcore_barrier()

    pltpu.sync_copy(shared_hist.at[pl.ds(s * RPT, RPT)],
                    hist_hbm.at[c].at[pl.ds(s * RPT, RPT)])


_hist_call = functools.partial(
    pl.kernel,
    out_type=jax.ShapeDtypeStruct((NSC, NP, 16), jnp.float32),
    mesh=_mesh,
    scratch_types=[
        pltpu.VMEM((CPT, CH), jnp.int32),
        pltpu.VMEM((CH, 16), jnp.float32),
        pltpu.VMEM((158, 16), jnp.float32),
        pltpu.VMEM_SHARED((NP, 16), jnp.float32),
    ],
)(_hist_body)


def _scat_body(u_hbm, z_hbm, srcp_hbm, dstp_hbm, s_hbm, dst_v, ring, gbuf0,
               gbuf1, shared_out, semg, sema, semb):
    c = lax.axis_index("c")
    s = lax.axis_index("s")
    w = c * NTS + s
    pltpu.sync_copy(dstp_hbm.at[w], dst_v)
    pltpu.sync_copy(z_hbm.at[pl.ds(s * RPT, RPT)],
                    shared_out.at[pl.ds(s * RPT, RPT)])
    sp = srcp_hbm.at[w]
    pltpu.sync_copy(sp.at[pl.ds(0, 2)], ring.at[pl.ds(0, 2)])
    pltpu.async_copy(sp.at[pl.ds(2, 2)], ring.at[pl.ds(2, 2)], semb)
    pltpu.async_copy(sp.at[pl.ds(4, 2)], ring.at[pl.ds(4, 2)], sema)
    plsc.subcore_barrier()

    pltpu.async_copy(u_hbm.at[ring.at[0]], gbuf0, semg).wait()

    def _half(p, wsem):
        j0 = 2 * p
        j1 = j0 + 1
        j2 = j0 + 2
        pltpu.make_async_copy(
            sp.at[pl.ds(0, 2)], ring.at[pl.ds(0, 2)], wsem).wait()
        jn = jnp.where(j0 + 6 < CPT, j0 + 6, 0)
        pltpu.async_copy(sp.at[pl.ds(jn, 2)],
                         ring.at[pl.ds(lax.rem(jn, 10), 2)], wsem)
        cp1 = pltpu.async_copy(u_hbm.at[ring.at[lax.rem(j1, 10)]], gbuf1,
                               semg)
        pltpu.sync_copy(gbuf0, shared_out.at[dst_v.at[j0]], add=True)
        cp1.wait()
        cp2 = pltpu.async_copy(u_hbm.at[ring.at[lax.rem(j2, 10)]], gbuf0,
                               semg)
        pltpu.sync_copy(gbuf1, shared_out.at[dst_v.at[j1]], add=True)
        cp2.wait()

    def _pairs(t, _):
        _half(2 * t, semb)
        _half(2 * t + 1, sema)
        return 0

    lax.fori_loop(0, 19, _pairs, 0)
    _half(38, semb)
    pltpu.sync_copy(gbuf0, shared_out.at[dst_v.at[CPT - 1]], add=True)
    pltpu.make_async_copy(sp.at[pl.ds(0, 2)], ring.at[pl.ds(0, 2)],
                          sema).wait()
    pltpu.make_async_copy(sp.at[pl.ds(0, 2)], ring.at[pl.ds(0, 2)],
                          semb).wait()
    plsc.subcore_barrier()

    pltpu.sync_copy(shared_out.at[pl.ds(s * RPT, RPT)],
                    s_hbm.at[c].at[pl.ds(s * RPT, RPT)])


_scat_call = functools.partial(
    pl.kernel,
    out_type=jax.ShapeDtypeStruct((NSC, NP, D), jnp.float32),
    mesh=_mesh,
    scratch_types=[
        pltpu.VMEM((CPT, CH), jnp.int32),
        pltpu.VMEM((10, CH), jnp.int32),
        pltpu.VMEM((CH, D), jnp.float32),
        pltpu.VMEM((CH, D), jnp.float32),
        pltpu.VMEM_SHARED((NP, D), jnp.float32),
        pltpu.SemaphoreType.DMA,
        pltpu.SemaphoreType.DMA,
        pltpu.SemaphoreType.DMA,
    ],
)(_scat_body)


def _tc1_body(hist_ref, x_ref, dinv_ref, u_ref):
    deg = hist_ref[0, :, 0:1] + hist_ref[1, :, 0:1] + 1.0
    dinv = lax.rsqrt(deg)
    dinv_ref[...] = dinv
    u_ref[...] = x_ref[...] * dinv


def _tc1(hist, xpad):
    return pl.pallas_call(
        _tc1_body,
        grid=(NB,),
        in_specs=[
            pl.BlockSpec((NSC, 128, 16), lambda r: (0, r, 0)),
            pl.BlockSpec((128, D), lambda r: (r, 0)),
        ],
        out_specs=[
            pl.BlockSpec((128, 1), lambda r: (r, 0)),
            pl.BlockSpec((128, D), lambda r: (r, 0)),
        ],
        out_shape=[
            jax.ShapeDtypeStruct((NP, 1), jnp.float32),
            jax.ShapeDtypeStruct((NP, D), jnp.float32),
        ],
    )(hist, xpad)


_SP_SPEC = pl.BlockSpec((NSC, 128, D), lambda r: (0, r, 0))
_ROW_SPEC = pl.BlockSpec((128, D), lambda r: (r, 0))
_DINV_SPEC = pl.BlockSpec((128, 1), lambda r: (r, 0))
_W_SPEC = pl.BlockSpec((D, D), lambda r: (0, 0))
_B_SPEC = pl.BlockSpec((1, D), lambda r: (0, 0))


def _tc2_body(sp_ref, u_ref, dinv_ref, o_ref):
    dinv = dinv_ref[...]
    d2 = dinv * dinv
    o_ref[...] = (sp_ref[0] + sp_ref[1] + u_ref[...]) * d2


def _tc2(sp, u, dinv):
    return pl.pallas_call(
        _tc2_body,
        grid=(NB,),
        in_specs=[_SP_SPEC, _ROW_SPEC, _DINV_SPEC],
        out_specs=_ROW_SPEC,
        out_shape=jax.ShapeDtypeStruct((NP, D), jnp.float32),
    )(sp, u, dinv)


def _tc3_body(sp_ref, u_ref, dinv_ref, Wsg_ref, bsg_ref, Wg1_ref, v_ref):
    dinv = dinv_ref[...]
    t = (sp_ref[0] + sp_ref[1] + u_ref[...]) * dinv
    h = jnp.maximum(
        jnp.dot(t, Wsg_ref[...], preferred_element_type=jnp.float32)
        + bsg_ref[...], 0.0)
    v_ref[...] = jnp.dot(
        h, Wg1_ref[...], preferred_element_type=jnp.float32) * dinv


def _tc3(sp, u2, dinv, W_sg, b_sg, W_g1):
    return pl.pallas_call(
        _tc3_body,
        grid=(NB,),
        in_specs=[_SP_SPEC, _ROW_SPEC, _DINV_SPEC, _W_SPEC, _B_SPEC, _W_SPEC],
        out_specs=_ROW_SPEC,
        out_shape=jax.ShapeDtypeStruct((NP, D), jnp.float32),
    )(sp, u2, dinv, W_sg, b_sg, W_g1)


def _tc45_body(sp_ref, v_ref, dinv_ref, b_ref, W_ref, o_ref):
    dinv = dinv_ref[...]
    t = (sp_ref[0] + sp_ref[1] + v_ref[...]) * dinv
    h = jnp.maximum(t + b_ref[...], 0.0)
    o_ref[...] = jnp.dot(
        h, W_ref[...], preferred_element_type=jnp.float32) * dinv


def _tc45(sp, v, dinv, b_prev, W_next):
    return pl.pallas_call(
        _tc45_body,
        grid=(NB,),
        in_specs=[_SP_SPEC, _ROW_SPEC, _DINV_SPEC, _B_SPEC, _W_SPEC],
        out_specs=_ROW_SPEC,
        out_shape=jax.ShapeDtypeStruct((NP, D), jnp.float32),
    )(sp, v, dinv, b_prev, W_next)


def _tc6_body(sp_ref, v_ref, dinv_ref, bg3_ref, batch_ref, W1_ref, b1_ref,
              W2_ref, b2_ref, W3_ref, b3_ref, out_ref, g_acc):
    r = pl.program_id(0)

    @pl.when(r == 0)
    def _():
        g_acc[...] = jnp.zeros_like(g_acc)

    dinv = dinv_ref[...]
    t = (sp_ref[0] + sp_ref[1] + v_ref[...]) * dinv
    h = jnp.maximum(t + bg3_ref[...], 0.0)
    gi = lax.broadcasted_iota(jnp.int32, (NG, 1), 0)
    oh = (batch_ref[0] == gi).astype(jnp.float32)
    g_acc[...] += jnp.dot(oh, h, preferred_element_type=jnp.float32)

    @pl.when(r == NB - 1)
    def _():
        g = g_acc[...]
        g1 = jnp.maximum(
            jnp.dot(g, W1_ref[...], preferred_element_type=jnp.float32)
            + b1_ref[...], 0.0)
        g2 = jnp.maximum(
            jnp.dot(g1, W2_ref[...], preferred_element_type=jnp.float32)
            + b2_ref[...], 0.0)
        out_ref[...] = (
            jnp.dot(g2, W3_ref[...], preferred_element_type=jnp.float32)
            + b3_ref[...])


def _tc6(sp, v3, dinv, b_g3, batchp, W1, b1, W2, b2, W3, b3):
    return pl.pallas_call(
        _tc6_body,
        grid=(NB,),
        in_specs=[
            _SP_SPEC, _ROW_SPEC, _DINV_SPEC, _B_SPEC,
            pl.BlockSpec((1, 1, 128), lambda r: (r, 0, 0)),
            _W_SPEC, _B_SPEC,
            pl.BlockSpec((D, DH), lambda r: (0, 0)),
            pl.BlockSpec((1, DH), lambda r: (0, 0)),
            pl.BlockSpec((DH, NCLS), lambda r: (0, 0)),
            pl.BlockSpec((1, NCLS), lambda r: (0, 0)),
        ],
        out_specs=pl.BlockSpec((NG, NCLS), lambda r: (0, 0)),
        out_shape=jax.ShapeDtypeStruct((NG, NCLS), jnp.float32),
        scratch_shapes=[pltpu.VMEM((NG, D), jnp.float32)],
    )(sp, v3, dinv, b_g3, batchp, W1, b1, W2, b2, W3, b3)


def kernel(x, edge_index, batch, W_sg, b_sg, W_g1, b_g1, W_g2, b_g2,
           W_g3, b_g3, W_fc1, b_fc1, W_fc2, b_fc2, W_fc3, b_fc3):
    src = edge_index[0]
    dst = edge_index[1]
    pad = EPAD - E
    srcp = jnp.concatenate(
        [src, jnp.zeros((pad,), jnp.int32)]).reshape(NTW, CPT, CH)
    srcp = jnp.pad(srcp, ((0, 0), (0, 1), (0, 0)))
    dstp = jnp.concatenate(
        [dst, jnp.full((pad,), N, jnp.int32)]).reshape(NTW, CPT, CH)
    xpad = jnp.pad(x, ((0, NP - N), (0, 0)))
    batchp = jnp.pad(batch, (0, NP - N),
                     constant_values=NG).reshape(NB, 1, 128)

    zeros = jnp.zeros((NP, D), jnp.float32)

    hist = _hist_call(dstp)
    dinv, u = _tc1(hist, xpad)
    sp1 = _scat_call(u, zeros, srcp, dstp)
    u2 = _tc2(sp1, u, dinv)
    sp2 = _scat_call(u2, zeros, srcp, dstp)
    v1 = _tc3(sp2, u2, dinv, W_sg, b_sg.reshape(1, D), W_g1)
    sp3 = _scat_call(v1, zeros, srcp, dstp)
    v2 = _tc45(sp3, v1, dinv, b_g1.reshape(1, D), W_g2)
    sp4 = _scat_call(v2, zeros, srcp, dstp)
    v3 = _tc45(sp4, v2, dinv, b_g2.reshape(1, D), W_g3)
    sp5 = _scat_call(v3, zeros, srcp, dstp)
    return _tc6(sp5, v3, dinv, b_g3.reshape(1, D), batchp,
                W_fc1, b_fc1.reshape(1, D), W_fc2, b_fc2.reshape(1, DH),
                W_fc3, b_fc3.reshape(1, NCLS))
